# Initial kernel scaffold; baseline (speedup 1.0000x reference)
import jax
import jax.numpy as jnp
from jax.experimental import pallas as pl


def kernel(x, edge_index, W1, b1, W2, b2, Wd1, bd1, Wd2, bd2):
    raise NotImplementedError("write your pallas kernel here")


# speedup vs baseline: 8.8996x; 8.8996x over previous
import functools

import jax
import jax.numpy as jnp
from jax import lax
from jax.experimental import pallas as pl
from jax.experimental.pallas import tpu as pltpu
from jax.experimental.pallas import tpu_sc as plsc

N_NODES = 10000
N_EDGES = 320000
D_IN = 128
D_HID = 128
D_OUT = 64

NC = 2
NS = 16
NW = NC * NS

N_PAD = 10240
E_PAD = 327680
EW = E_PAD // NW
CHUNK = 128
NCHUNK = EW // CHUNK
RPS = N_PAD // NS

PAD_SRC = 10000
PAD_DST = 10016

_mesh = plsc.VectorSubcoreMesh(
    core_axis_name="c", subcore_axis_name="s", num_cores=NC, num_subcores=NS
)


def _deg_body(dst_hbm, zeros_hbm, out_hbm, dst_v, ones_v, acc_sh):
    cid = lax.axis_index("c")
    sid = lax.axis_index("s")
    wid = cid * NS + sid
    pltpu.sync_copy(
        zeros_hbm.at[pl.ds(sid * RPS, RPS)], acc_sh.at[pl.ds(sid * RPS, RPS)]
    )
    for j in range(CHUNK // 16):
        ones_v[pl.ds(j * 16, 16)] = jnp.full((16,), 1.0, jnp.float32)
    plsc.subcore_barrier()

    base = wid * EW

    def step(i, carry):
        off = base + i * CHUNK
        pltpu.sync_copy(dst_hbm.at[pl.ds(off, CHUNK)], dst_v)
        pltpu.sync_copy(ones_v, acc_sh.at[dst_v], add=True)
        return carry

    lax.fori_loop(0, NCHUNK, step, 0)
    plsc.subcore_barrier()
    pltpu.sync_copy(
        acc_sh.at[pl.ds(sid * RPS, RPS)], out_hbm.at[cid, pl.ds(sid * RPS, RPS)]
    )


_deg_kernel = functools.partial(
    pl.kernel,
    out_type=jax.ShapeDtypeStruct((NC, N_PAD), jnp.float32),
    mesh=_mesh,
    scratch_types=[
        pltpu.VMEM((CHUNK,), jnp.int32),
        pltpu.VMEM((CHUNK,), jnp.float32),
        pltpu.VMEM_SHARED((N_PAD,), jnp.float32),
    ],
)(_deg_body)


def _conv_body(h_hbm, src_hbm, dst_hbm, zeros_hbm, out_hbm,
               src_v, dst_v, rows_v, acc_sh, sem):
    cid = lax.axis_index("c")
    sid = lax.axis_index("s")
    wid = cid * NS + sid
    pltpu.sync_copy(
        zeros_hbm.at[pl.ds(sid * RPS, RPS)], acc_sh.at[pl.ds(sid * RPS, RPS)]
    )
    plsc.subcore_barrier()

    base = wid * EW

    def step(i, carry):
        off = base + i * CHUNK
        pltpu.sync_copy(src_hbm.at[pl.ds(off, CHUNK)], src_v)
        pltpu.sync_copy(dst_hbm.at[pl.ds(off, CHUNK)], dst_v)
        pltpu.async_copy(h_hbm.at[src_v], rows_v, sem).wait()
        pltpu.sync_copy(rows_v, acc_sh.at[dst_v], add=True)
        return carry

    lax.fori_loop(0, NCHUNK, step, 0)
    plsc.subcore_barrier()
    pltpu.sync_copy(
        acc_sh.at[pl.ds(sid * RPS, RPS)],
        out_hbm.at[cid, pl.ds(sid * RPS, RPS)],
    )


def _make_conv_kernel(d):
    return functools.partial(
        pl.kernel,
        out_type=jax.ShapeDtypeStruct((NC, N_PAD, d), jnp.float32),
        mesh=_mesh,
        compiler_params=pltpu.CompilerParams(use_tc_tiling_on_sc=(d == 128)),
        scratch_types=[
            pltpu.VMEM((CHUNK,), jnp.int32),
            pltpu.VMEM((CHUNK,), jnp.int32),
            pltpu.VMEM((CHUNK, d), jnp.float32),
            pltpu.VMEM_SHARED((N_PAD, d), jnp.float32),
            pltpu.SemaphoreType.DMA,
        ],
    )(_conv_body)


_conv128 = _make_conv_kernel(D_HID)
_conv64 = _make_conv_kernel(D_OUT)


BR = 1280
GRID = N_PAD // BR


def _dm(pid, deg0, deg1):
    rows = lax.broadcasted_iota(jnp.int32, (BR, 1), 0) + pid * BR
    deg = deg0[...] + deg1[...] + 1.0
    return jnp.where(rows < N_NODES, lax.rsqrt(deg), 0.0)


def _mm1_body(x_ref, w_ref, deg0, deg1, o_ref):
    dm = _dm(pl.program_id(0), deg0, deg1)
    o_ref[...] = jnp.dot(
        x_ref[...], w_ref[...], preferred_element_type=jnp.float32
    ) * dm


def _mid_body(a0, a1, hp, deg0, deg1, b1, w2, o_ref):
    dm = _dm(pl.program_id(0), deg0, deg1)
    z1 = jnp.maximum((a0[...] + a1[...] + hp[...]) * dm + b1[...], 0.0)
    o_ref[...] = jnp.dot(z1, w2[...], preferred_element_type=jnp.float32) * dm


def _fin_body(a0, a1, hp, deg0, deg1, b2, wd1, bd1, wd2, bd2, o_ref):
    dm = _dm(pl.program_id(0), deg0, deg1)
    z2 = jnp.maximum((a0[...] + a1[...] + hp[...]) * dm + b2[...], 0.0)
    z3 = jnp.maximum(
        jnp.dot(z2, wd1[...], preferred_element_type=jnp.float32) + bd1[...], 0.0
    )
    o_ref[...] = (
        jnp.dot(z3, wd2[...], preferred_element_type=jnp.float32) + bd2[...]
    )


def _row_spec(d):
    return pl.BlockSpec((BR, d), lambda i: (i, 0))


def _full_spec(r, c):
    return pl.BlockSpec((r, c), lambda i: (0, 0))


_deg_spec = pl.BlockSpec((BR, 1), lambda i: (i, 0))


def _mm1(xp, W1, deg0, deg1):
    return pl.pallas_call(
        _mm1_body,
        grid=(GRID,),
        in_specs=[_row_spec(D_IN), _full_spec(D_IN, D_HID), _deg_spec, _deg_spec],
        out_specs=_row_spec(D_HID),
        out_shape=jax.ShapeDtypeStruct((N_PAD, D_HID), jnp.float32),
    )(xp, W1, deg0, deg1)


def _mid(a0, a1, hp, deg0, deg1, b1, W2):
    return pl.pallas_call(
        _mid_body,
        grid=(GRID,),
        in_specs=[
            _row_spec(D_HID), _row_spec(D_HID), _row_spec(D_HID),
            _deg_spec, _deg_spec,
            _full_spec(1, D_HID), _full_spec(D_HID, D_OUT),
        ],
        out_specs=_row_spec(D_OUT),
        out_shape=jax.ShapeDtypeStruct((N_PAD, D_OUT), jnp.float32),
    )(a0, a1, hp, deg0, deg1, b1, W2)


def _fin(a0, a1, hp, deg0, deg1, b2, Wd1, bd1, Wd2, bd2):
    return pl.pallas_call(
        _fin_body,
        grid=(GRID,),
        in_specs=[
            _row_spec(D_OUT), _row_spec(D_OUT), _row_spec(D_OUT),
            _deg_spec, _deg_spec,
            _full_spec(1, D_OUT), _full_spec(D_OUT, D_HID),
            _full_spec(1, D_HID), _full_spec(D_HID, D_IN), _full_spec(1, D_IN),
        ],
        out_specs=_row_spec(D_IN),
        out_shape=jax.ShapeDtypeStruct((N_PAD, D_IN), jnp.float32),
    )(a0, a1, hp, deg0, deg1, b2, Wd1, bd1, Wd2, bd2)


def kernel(x, edge_index, W1, b1, W2, b2, Wd1, bd1, Wd2, bd2):
    src = edge_index[0].astype(jnp.int32)
    dst = edge_index[1].astype(jnp.int32)
    pad_e = E_PAD - N_EDGES
    srcp = jnp.concatenate([src, jnp.full((pad_e,), PAD_SRC, jnp.int32)])
    dstp = jnp.concatenate([dst, jnp.full((pad_e,), PAD_DST, jnp.int32)])

    xp = jnp.zeros((N_PAD, D_IN), jnp.float32).at[:N_NODES].set(x)
    zeros1 = jnp.zeros((N_PAD,), jnp.float32)
    zerosA = jnp.zeros((N_PAD, D_HID), jnp.float32)
    zerosB = jnp.zeros((N_PAD, D_OUT), jnp.float32)

    degp = _deg_kernel(dstp, zeros1)
    deg0 = degp[0][:, None]
    deg1 = degp[1][:, None]

    h1p = _mm1(xp, W1, deg0, deg1)
    acc1 = _conv128(h1p, srcp, dstp, zerosA)
    h2p = _mid(acc1[0], acc1[1], h1p, deg0, deg1,
               b1[None, :], W2)
    acc2 = _conv64(h2p, srcp, dstp, zerosB)
    xh = _fin(acc2[0], acc2[1], h2p, deg0, deg1,
              b2[None, :], Wd1, bd1[None, :], Wd2, bd2[None, :])
    return xh[:N_NODES]


# speedup vs baseline: 13.2253x; 1.4861x over previous
import functools

import jax
import jax.numpy as jnp
from jax import lax
from jax.experimental import pallas as pl
from jax.experimental.pallas import tpu as pltpu
from jax.experimental.pallas import tpu_sc as plsc

N_NODES = 10000
N_EDGES = 320000
D_IN = 128
D_HID = 128
D_OUT = 64

NC = 2
NS = 16
NW = NC * NS

N_PAD = 10240
E_PAD = 327680
EW = E_PAD // NW
CHUNK = 128
NCHUNK = EW // CHUNK
RPS = N_PAD // NS

PAD_SRC = 10000
PAD_DST = 10016

_mesh = plsc.VectorSubcoreMesh(
    core_axis_name="c", subcore_axis_name="s", num_cores=NC, num_subcores=NS
)


def _deg_body(dst_hbm, zeros_hbm, out_hbm, dst_v, ones_v, acc_sh):
    cid = lax.axis_index("c")
    sid = lax.axis_index("s")
    wid = cid * NS + sid
    pltpu.sync_copy(
        zeros_hbm.at[pl.ds(sid * RPS, RPS)], acc_sh.at[pl.ds(sid * RPS, RPS)]
    )
    for j in range(CHUNK // 16):
        ones_v[pl.ds(j * 16, 16)] = jnp.full((16,), 1.0, jnp.float32)
    plsc.subcore_barrier()

    base = wid * EW

    def step(i, carry):
        off = base + i * CHUNK
        pltpu.sync_copy(dst_hbm.at[pl.ds(off, CHUNK)], dst_v)
        pltpu.sync_copy(ones_v, acc_sh.at[dst_v], add=True)
        return carry

    lax.fori_loop(0, NCHUNK, step, 0)
    plsc.subcore_barrier()
    pltpu.sync_copy(
        acc_sh.at[pl.ds(sid * RPS, RPS)], out_hbm.at[cid, pl.ds(sid * RPS, RPS)]
    )


_deg_kernel = functools.partial(
    pl.kernel,
    out_type=jax.ShapeDtypeStruct((NC, N_PAD), jnp.float32),
    mesh=_mesh,
    scratch_types=[
        pltpu.VMEM((CHUNK,), jnp.int32),
        pltpu.VMEM((CHUNK,), jnp.float32),
        pltpu.VMEM_SHARED((N_PAD,), jnp.float32),
    ],
)(_deg_body)


def _conv_body(nbuf, h_hbm, sd_hbm, zeros_hbm, out_hbm,
               idx_v, rows_v, acc_sh, sem_i, sem_g):
    NBUF = nbuf
    NIBUF = 2 * NBUF
    STEADY = NCHUNK // NBUF - 2
    cid = lax.axis_index("c")
    sid = lax.axis_index("s")
    wid = cid * NS + sid
    pltpu.sync_copy(
        zeros_hbm.at[pl.ds(sid * RPS, RPS)], acc_sh.at[pl.ds(sid * RPS, RPS)]
    )

    cbase = wid * NCHUNK

    def start_idx(islot, chunk):
        pltpu.async_copy(sd_hbm.at[cbase + chunk], idx_v.at[islot], sem_i.at[islot])

    def wait_idx(islot):
        pltpu.make_async_copy(
            sd_hbm.at[cbase], idx_v.at[islot], sem_i.at[islot]
        ).wait()

    def start_gather(b, islot):
        pltpu.async_copy(h_hbm.at[idx_v.at[islot, 0]], rows_v.at[b], sem_g.at[b])

    def wait_gather(b):
        pltpu.make_async_copy(
            h_hbm.at[pl.ds(0, CHUNK)], rows_v.at[b], sem_g.at[b]
        ).wait()

    def scatter(b, islot):
        pltpu.sync_copy(rows_v.at[b], acc_sh.at[idx_v.at[islot, 1]], add=True)

    for c in range(NIBUF):
        start_idx(c, c)
    plsc.subcore_barrier()
    for b in range(NBUF):
        wait_idx(b)
        start_gather(b, b)

    def steady(g, carry):
        i0 = g * NBUF
        for b in range(NBUF):
            i = i0 + b
            wait_gather(b)
            scatter(b, i % NIBUF)
            start_idx(i % NIBUF, i + NIBUF)
            wait_idx((i + NBUF) % NIBUF)
            start_gather(b, (i + NBUF) % NIBUF)
        return carry

    lax.fori_loop(0, STEADY, steady, 0, unroll=False)

    i0 = STEADY * NBUF
    for b in range(NBUF):
        i = i0 + b
        wait_gather(b)
        scatter(b, i % NIBUF)
        wait_idx((i + NBUF) % NIBUF)
        start_gather(b, (i + NBUF) % NIBUF)
    i0 += NBUF
    for b in range(NBUF):
        i = i0 + b
        wait_gather(b)
        scatter(b, i % NIBUF)

    plsc.subcore_barrier()
    pltpu.sync_copy(
        acc_sh.at[pl.ds(sid * RPS, RPS)],
        out_hbm.at[cid, pl.ds(sid * RPS, RPS)],
    )


def _make_conv_kernel(d, nbuf):
    return functools.partial(
        pl.kernel,
        out_type=jax.ShapeDtypeStruct((NC, N_PAD, d), jnp.float32),
        mesh=_mesh,
        compiler_params=pltpu.CompilerParams(use_tc_tiling_on_sc=(d == 128)),
        scratch_types=[
            pltpu.VMEM((2 * nbuf, 2, CHUNK), jnp.int32),
            pltpu.VMEM((nbuf, CHUNK, d), jnp.float32),
            pltpu.VMEM_SHARED((N_PAD, d), jnp.float32),
            pltpu.SemaphoreType.DMA((2 * nbuf,)),
            pltpu.SemaphoreType.DMA((nbuf,)),
        ],
    )(functools.partial(_conv_body, nbuf))


_conv128 = _make_conv_kernel(D_HID, 2)
_conv64 = _make_conv_kernel(D_OUT, 4)


BR = 1280
GRID = N_PAD // BR


def _dm(pid, deg0, deg1):
    rows = lax.broadcasted_iota(jnp.int32, (BR, 1), 0) + pid * BR
    deg = deg0[...] + deg1[...] + 1.0
    return jnp.where(rows < N_NODES, lax.rsqrt(deg), 0.0)


def _mm1_body(x_ref, w_ref, deg0, deg1, o_ref):
    dm = _dm(pl.program_id(0), deg0, deg1)
    o_ref[...] = jnp.dot(
        x_ref[...], w_ref[...], preferred_element_type=jnp.float32
    ) * dm


def _mid_body(a0, a1, hp, deg0, deg1, b1, w2, o_ref):
    dm = _dm(pl.program_id(0), deg0, deg1)
    z1 = jnp.maximum((a0[...] + a1[...] + hp[...]) * dm + b1[...], 0.0)
    o_ref[...] = jnp.dot(z1, w2[...], preferred_element_type=jnp.float32) * dm


def _fin_body(a0, a1, hp, deg0, deg1, b2, wd1, bd1, wd2, bd2, o_ref):
    dm = _dm(pl.program_id(0), deg0, deg1)
    z2 = jnp.maximum((a0[...] + a1[...] + hp[...]) * dm + b2[...], 0.0)
    z3 = jnp.maximum(
        jnp.dot(z2, wd1[...], preferred_element_type=jnp.float32) + bd1[...], 0.0
    )
    o_ref[...] = (
        jnp.dot(z3, wd2[...], preferred_element_type=jnp.float32) + bd2[...]
    )


def _row_spec(d):
    return pl.BlockSpec((BR, d), lambda i: (i, 0))


def _full_spec(r, c):
    return pl.BlockSpec((r, c), lambda i: (0, 0))


_deg_spec = pl.BlockSpec((BR, 1), lambda i: (i, 0))


def _mm1(xp, W1, deg0, deg1):
    return pl.pallas_call(
        _mm1_body,
        grid=(GRID,),
        in_specs=[_row_spec(D_IN), _full_spec(D_IN, D_HID), _deg_spec, _deg_spec],
        out_specs=_row_spec(D_HID),
        out_shape=jax.ShapeDtypeStruct((N_PAD, D_HID), jnp.float32),
    )(xp, W1, deg0, deg1)


def _mid(a0, a1, hp, deg0, deg1, b1, W2):
    return pl.pallas_call(
        _mid_body,
        grid=(GRID,),
        in_specs=[
            _row_spec(D_HID), _row_spec(D_HID), _row_spec(D_HID),
            _deg_spec, _deg_spec,
            _full_spec(1, D_HID), _full_spec(D_HID, D_OUT),
        ],
        out_specs=_row_spec(D_OUT),
        out_shape=jax.ShapeDtypeStruct((N_PAD, D_OUT), jnp.float32),
    )(a0, a1, hp, deg0, deg1, b1, W2)


def _fin(a0, a1, hp, deg0, deg1, b2, Wd1, bd1, Wd2, bd2):
    return pl.pallas_call(
        _fin_body,
        grid=(GRID,),
        in_specs=[
            _row_spec(D_OUT), _row_spec(D_OUT), _row_spec(D_OUT),
            _deg_spec, _deg_spec,
            _full_spec(1, D_OUT), _full_spec(D_OUT, D_HID),
            _full_spec(1, D_HID), _full_spec(D_HID, D_IN), _full_spec(1, D_IN),
        ],
        out_specs=_row_spec(D_IN),
        out_shape=jax.ShapeDtypeStruct((N_PAD, D_IN), jnp.float32),
    )(a0, a1, hp, deg0, deg1, b2, Wd1, bd1, Wd2, bd2)


def kernel(x, edge_index, W1, b1, W2, b2, Wd1, bd1, Wd2, bd2):
    src = edge_index[0].astype(jnp.int32)
    dst = edge_index[1].astype(jnp.int32)
    pad_e = E_PAD - N_EDGES
    srcp = jnp.concatenate([src, jnp.full((pad_e,), PAD_SRC, jnp.int32)])
    dstp = jnp.concatenate([dst, jnp.full((pad_e,), PAD_DST, jnp.int32)])
    sd = jnp.stack(
        [srcp.reshape(-1, CHUNK), dstp.reshape(-1, CHUNK)], axis=1
    )

    xp = jnp.zeros((N_PAD, D_IN), jnp.float32).at[:N_NODES].set(x)
    zeros1 = jnp.zeros((N_PAD,), jnp.float32)
    zerosA = jnp.zeros((N_PAD, D_HID), jnp.float32)
    zerosB = jnp.zeros((N_PAD, D_OUT), jnp.float32)

    degp = _deg_kernel(dstp, zeros1)
    deg0 = degp[0][:, None]
    deg1 = degp[1][:, None]

    h1p = _mm1(xp, W1, deg0, deg1)
    acc1 = _conv128(h1p, sd, zerosA)
    h2p = _mid(acc1[0], acc1[1], h1p, deg0, deg1,
               b1[None, :], W2)
    acc2 = _conv64(h2p, sd, zerosB)
    xh = _fin(acc2[0], acc2[1], h2p, deg0, deg1,
              b2[None, :], Wd1, bd1[None, :], Wd2, bd2[None, :])
    return xh[:N_NODES]
